# Initial kernel scaffold; baseline (speedup 1.0000x reference)
import jax
import jax.numpy as jnp
from jax.experimental import pallas as pl


def kernel(x, edge_index, W1, b1, W2, b2):
    raise NotImplementedError("write your pallas kernel here")


# speedup vs baseline: 14.3514x; 14.3514x over previous
import functools

import jax
import jax.numpy as jnp
from jax import lax
from jax.experimental import pallas as pl
from jax.experimental.pallas import tpu as pltpu
from jax.experimental.pallas import tpu_sc as plsc

NC = 2
NS = 16
NW = NC * NS
B = 128


def _largest_divisor_le(n, cap):
    for d in range(cap, 0, -1):
        if n % d == 0:
            return d
    return 1


def _deg_body(nrows, cpw, zr, dst_ref, degp_ref, acc, didx, ones_v, zbuf):
    rps = nrows // NS
    nz = rps // zr
    c = lax.axis_index("c")
    s = lax.axis_index("s")
    w = s * NC + c

    def oloop(i, carry):
        ones_v[i] = jnp.full((16,), 1.0, jnp.float32)
        return carry
    lax.fori_loop(0, B, oloop, 0)

    def zloop(i, carry):
        zbuf[i] = jnp.zeros((16,), jnp.float32)
        return carry
    lax.fori_loop(0, zr, zloop, 0)
    for t in range(nz):
        pltpu.sync_copy(zbuf, acc.at[pl.ds(s * rps + t * zr, zr)])
    plsc.subcore_barrier()

    def body(k, carry):
        pltpu.sync_copy(dst_ref.at[pl.ds((w * cpw + k) * B, B)], didx)
        pltpu.sync_copy(ones_v, acc.at[didx], add=True)
        return carry
    lax.fori_loop(0, cpw, body, 0)
    plsc.subcore_barrier()

    pltpu.sync_copy(acc.at[pl.ds(s * rps, rps)],
                    degp_ref.at[c, pl.ds(s * rps, rps)])


def _edge_body(nrows, d, cpw, zr, g_ref, src_ref, dst_ref, p_ref,
               acc, sidx, didx, rows0, zbuf, sem0):
    rps = nrows // NS
    nz = rps // zr
    vpr = d // 16
    c = lax.axis_index("c")
    s = lax.axis_index("s")
    w = s * NC + c

    def zloop(t, carry):
        i = t // vpr
        j = t - i * vpr
        zbuf[i, pl.ds(j * 16, 16)] = jnp.zeros((16,), jnp.float32)
        return carry
    lax.fori_loop(0, zr * vpr, zloop, 0)
    for t in range(nz):
        pltpu.sync_copy(zbuf, acc.at[pl.ds(s * rps + t * zr, zr)])
    plsc.subcore_barrier()

    def body(k, carry):
        base = (w * cpw + k) * B
        pltpu.sync_copy(src_ref.at[pl.ds(base, B)], sidx)
        pltpu.sync_copy(dst_ref.at[pl.ds(base, B)], didx)
        pltpu.async_copy(g_ref.at[sidx], rows0, sem0).wait()
        pltpu.sync_copy(rows0, acc.at[didx], add=True)
        return carry
    lax.fori_loop(0, cpw, body, 0)
    plsc.subcore_barrier()

    pltpu.sync_copy(acc.at[pl.ds(s * rps, rps)],
                    p_ref.at[c, pl.ds(s * rps, rps)])


def _tc1_body(degs_ref, x_ref, w_ref, dinv_ref, g_ref):
    deg = degs_ref[:, 0:1] + degs_ref[:, 1:2] + 1.0
    dinv = lax.rsqrt(deg)
    dinv_ref[...] = dinv
    g_ref[...] = jnp.dot(x_ref[...], w_ref[...],
                         preferred_element_type=jnp.float32,
                         precision=lax.Precision.HIGHEST) * dinv


def _tc2_body(p_ref, g1_ref, dinv_ref, b_ref, w_ref, g2_ref):
    ssum = p_ref[0] + p_ref[1] + g1_ref[...]
    h = jnp.maximum(dinv_ref[...] * ssum + b_ref[...], 0.0)
    g2_ref[...] = jnp.dot(h, w_ref[...],
                          preferred_element_type=jnp.float32,
                          precision=lax.Precision.HIGHEST) * dinv_ref[...]


def _tc3_body(p_ref, g2_ref, dinv_ref, b_ref, out_ref):
    out_ref[...] = dinv_ref[...] * (p_ref[0] + p_ref[1] + g2_ref[...]) \
        + b_ref[...]


def kernel(x, edge_index, W1, b1, W2, b2):
    n, d_in = x.shape
    d_hid = W1.shape[1]
    d_out = W2.shape[1]
    e = edge_index.shape[1]

    cpw = -(-e // (NW * B))
    cpw += cpw % 2
    pe = cpw * NW * B
    pad = pe - e
    nrows = ((n + 127) // 128 + 1) * 128
    ndump = nrows - n
    assert nrows % (NS * 8) == 0 and d_in % 16 == 0
    zr = _largest_divisor_le(nrows // NS, 128)
    zre = _largest_divisor_le(nrows // NS, 32)

    src = edge_index[0].astype(jnp.int32)
    dst = edge_index[1].astype(jnp.int32)
    ar = jnp.arange(pad, dtype=jnp.int32)
    src2 = jnp.concatenate([src, (ar * 997) % n])
    dst2 = jnp.concatenate([dst, n + (ar % ndump)])

    mesh = plsc.VectorSubcoreMesh(core_axis_name="c", subcore_axis_name="s")

    deg_call = pl.kernel(
        functools.partial(_deg_body, nrows, cpw, zr),
        out_type=jax.ShapeDtypeStruct((NC, nrows, 16), jnp.float32),
        mesh=mesh,
        scratch_types=[
            pltpu.VMEM_SHARED((nrows, 16), jnp.float32),
            pltpu.VMEM((B,), jnp.int32),
            pltpu.VMEM((B, 16), jnp.float32),
            pltpu.VMEM((zr, 16), jnp.float32),
        ],
    )
    degp = deg_call(dst2)
    degs = jnp.transpose(degp[:, :n, 0])

    def edge_call(d, g):
        return pl.kernel(
            functools.partial(_edge_body, nrows, d, cpw, zre),
            out_type=jax.ShapeDtypeStruct((NC, nrows, d), jnp.float32),
            mesh=mesh,
            scratch_types=[
                pltpu.VMEM_SHARED((nrows, d), jnp.float32),
                pltpu.VMEM((B,), jnp.int32),
                pltpu.VMEM((B,), jnp.int32),
                pltpu.VMEM((B, d), jnp.float32),
                pltpu.VMEM((zre, d), jnp.float32),
                pltpu.SemaphoreType.DMA,
            ],
        )(g, src2, dst2)

    r = n // 10
    dinv, g1 = pl.pallas_call(
        _tc1_body,
        grid=(n // r,),
        in_specs=[
            pl.BlockSpec((r, 2), lambda i: (i, 0)),
            pl.BlockSpec((r, d_in), lambda i: (i, 0)),
            pl.BlockSpec((d_in, d_hid), lambda i: (0, 0)),
        ],
        out_specs=[
            pl.BlockSpec((r, 1), lambda i: (i, 0)),
            pl.BlockSpec((r, d_hid), lambda i: (i, 0)),
        ],
        out_shape=[
            jax.ShapeDtypeStruct((n, 1), jnp.float32),
            jax.ShapeDtypeStruct((n, d_hid), jnp.float32),
        ],
    )(degs, x, W1)

    p1 = edge_call(d_hid, g1)

    g2 = pl.pallas_call(
        _tc2_body,
        grid=(n // r,),
        in_specs=[
            pl.BlockSpec((NC, r, d_hid), lambda i: (0, i, 0)),
            pl.BlockSpec((r, d_hid), lambda i: (i, 0)),
            pl.BlockSpec((r, 1), lambda i: (i, 0)),
            pl.BlockSpec((1, d_hid), lambda i: (0, 0)),
            pl.BlockSpec((d_hid, d_out), lambda i: (0, 0)),
        ],
        out_specs=pl.BlockSpec((r, d_out), lambda i: (i, 0)),
        out_shape=jax.ShapeDtypeStruct((n, d_out), jnp.float32),
    )(p1, g1, dinv, b1.reshape(1, -1), W2)

    p2 = edge_call(d_out, g2)

    out = pl.pallas_call(
        _tc3_body,
        grid=(n // r,),
        in_specs=[
            pl.BlockSpec((NC, r, d_out), lambda i: (0, i, 0)),
            pl.BlockSpec((r, d_out), lambda i: (i, 0)),
            pl.BlockSpec((r, 1), lambda i: (i, 0)),
            pl.BlockSpec((1, d_out), lambda i: (0, 0)),
        ],
        out_specs=pl.BlockSpec((r, d_out), lambda i: (i, 0)),
        out_shape=jax.ShapeDtypeStruct((n, d_out), jnp.float32),
    )(p2, g2, dinv, b2.reshape(1, -1))

    return out


# speedup vs baseline: 18.1219x; 1.2627x over previous
import functools

import jax
import jax.numpy as jnp
from jax import lax
from jax.experimental import pallas as pl
from jax.experimental.pallas import tpu as pltpu
from jax.experimental.pallas import tpu_sc as plsc

NC = 2
NS = 16
NW = NC * NS
B = 128


def _largest_divisor_le(n, cap):
    for d in range(cap, 0, -1):
        if n % d == 0:
            return d
    return 1


def _deg_body(nrows, cpw, zr, dst_ref, degp_ref, acc, didx, ones_v, zbuf):
    rps = nrows // NS
    nz = rps // zr
    c = lax.axis_index("c")
    s = lax.axis_index("s")
    w = s * NC + c

    def oloop(i, carry):
        ones_v[i] = jnp.full((16,), 1.0, jnp.float32)
        return carry
    lax.fori_loop(0, B, oloop, 0)

    def zloop(i, carry):
        zbuf[i] = jnp.zeros((16,), jnp.float32)
        return carry
    lax.fori_loop(0, zr, zloop, 0)
    for t in range(nz):
        pltpu.sync_copy(zbuf, acc.at[pl.ds(s * rps + t * zr, zr)])
    plsc.subcore_barrier()

    def body(k, carry):
        pltpu.sync_copy(dst_ref.at[pl.ds((w * cpw + k) * B, B)], didx)
        pltpu.sync_copy(ones_v, acc.at[didx], add=True)
        return carry
    lax.fori_loop(0, cpw, body, 0)
    plsc.subcore_barrier()

    pltpu.sync_copy(acc.at[pl.ds(s * rps, rps)],
                    degp_ref.at[c, pl.ds(s * rps, rps)])


def _edge_body(nrows, d, cpw, zr, g_ref, src_ref, dst_ref, p_ref,
               acc, sidx0, didx0, sidx1, didx1, rows0, rows1, zbuf,
               sem0, sem1):
    rps = nrows // NS
    nz = rps // zr
    vpr = d // 16
    c = lax.axis_index("c")
    s = lax.axis_index("s")
    w = s * NC + c

    def zloop(t, carry):
        i = t // vpr
        j = t - i * vpr
        zbuf[i, pl.ds(j * 16, 16)] = jnp.zeros((16,), jnp.float32)
        return carry
    lax.fori_loop(0, zr * vpr, zloop, 0)
    for t in range(nz):
        pltpu.sync_copy(zbuf, acc.at[pl.ds(s * rps + t * zr, zr)])
    plsc.subcore_barrier()

    def body(j, carry):
        base0 = (w * cpw + 2 * j) * B
        pltpu.sync_copy(src_ref.at[pl.ds(base0, B)], sidx0)
        pltpu.sync_copy(dst_ref.at[pl.ds(base0, B)], didx0)
        d0 = pltpu.async_copy(g_ref.at[sidx0], rows0, sem0)
        pltpu.sync_copy(src_ref.at[pl.ds(base0 + B, B)], sidx1)
        pltpu.sync_copy(dst_ref.at[pl.ds(base0 + B, B)], didx1)
        d1 = pltpu.async_copy(g_ref.at[sidx1], rows1, sem1)
        d0.wait()
        pltpu.sync_copy(rows0, acc.at[didx0], add=True)
        d1.wait()
        pltpu.sync_copy(rows1, acc.at[didx1], add=True)
        return carry
    lax.fori_loop(0, cpw // 2, body, 0)
    plsc.subcore_barrier()

    pltpu.sync_copy(acc.at[pl.ds(s * rps, rps)],
                    p_ref.at[c, pl.ds(s * rps, rps)])


def _tc1_body(degs_ref, x_ref, w_ref, dinv_ref, g_ref):
    deg = degs_ref[:, 0:1] + degs_ref[:, 1:2] + 1.0
    dinv = lax.rsqrt(deg)
    dinv_ref[...] = dinv
    g_ref[...] = jnp.dot(x_ref[...], w_ref[...],
                         preferred_element_type=jnp.float32,
                         precision=lax.Precision.HIGHEST) * dinv


def _tc2_body(p_ref, g1_ref, dinv_ref, b_ref, w_ref, g2_ref):
    ssum = p_ref[0] + p_ref[1] + g1_ref[...]
    h = jnp.maximum(dinv_ref[...] * ssum + b_ref[...], 0.0)
    g2_ref[...] = jnp.dot(h, w_ref[...],
                          preferred_element_type=jnp.float32,
                          precision=lax.Precision.HIGHEST) * dinv_ref[...]


def _tc3_body(p_ref, g2_ref, dinv_ref, b_ref, out_ref):
    out_ref[...] = dinv_ref[...] * (p_ref[0] + p_ref[1] + g2_ref[...]) \
        + b_ref[...]


def kernel(x, edge_index, W1, b1, W2, b2):
    n, d_in = x.shape
    d_hid = W1.shape[1]
    d_out = W2.shape[1]
    e = edge_index.shape[1]

    cpw = -(-e // (NW * B))
    cpw += cpw % 2
    pe = cpw * NW * B
    pad = pe - e
    nrows = ((n + 127) // 128 + 1) * 128
    ndump = nrows - n
    assert nrows % (NS * 8) == 0 and d_in % 16 == 0
    zr = _largest_divisor_le(nrows // NS, 128)
    zre = _largest_divisor_le(nrows // NS, 32)

    src = edge_index[0].astype(jnp.int32)
    dst = edge_index[1].astype(jnp.int32)
    ar = jnp.arange(pad, dtype=jnp.int32)
    src2 = jnp.concatenate([src, (ar * 997) % n])
    dst2 = jnp.concatenate([dst, n + (ar % ndump)])

    mesh = plsc.VectorSubcoreMesh(core_axis_name="c", subcore_axis_name="s")

    deg_call = pl.kernel(
        functools.partial(_deg_body, nrows, cpw, zr),
        out_type=jax.ShapeDtypeStruct((NC, nrows, 16), jnp.float32),
        mesh=mesh,
        scratch_types=[
            pltpu.VMEM_SHARED((nrows, 16), jnp.float32),
            pltpu.VMEM((B,), jnp.int32),
            pltpu.VMEM((B, 16), jnp.float32),
            pltpu.VMEM((zr, 16), jnp.float32),
        ],
    )
    degp = deg_call(dst2)
    degs = jnp.transpose(degp[:, :n, 0])

    def edge_call(d, g):
        return pl.kernel(
            functools.partial(_edge_body, nrows, d, cpw, zre),
            out_type=jax.ShapeDtypeStruct((NC, nrows, d), jnp.float32),
            mesh=mesh,
            scratch_types=[
                pltpu.VMEM_SHARED((nrows, d), jnp.float32),
                pltpu.VMEM((B,), jnp.int32),
                pltpu.VMEM((B,), jnp.int32),
                pltpu.VMEM((B,), jnp.int32),
                pltpu.VMEM((B,), jnp.int32),
                pltpu.VMEM((B, d), jnp.float32),
                pltpu.VMEM((B, d), jnp.float32),
                pltpu.VMEM((zre, d), jnp.float32),
                pltpu.SemaphoreType.DMA,
                pltpu.SemaphoreType.DMA,
            ],
        )(g, src2, dst2)

    r = n // 10
    dinv, g1 = pl.pallas_call(
        _tc1_body,
        grid=(n // r,),
        in_specs=[
            pl.BlockSpec((r, 2), lambda i: (i, 0)),
            pl.BlockSpec((r, d_in), lambda i: (i, 0)),
            pl.BlockSpec((d_in, d_hid), lambda i: (0, 0)),
        ],
        out_specs=[
            pl.BlockSpec((r, 1), lambda i: (i, 0)),
            pl.BlockSpec((r, d_hid), lambda i: (i, 0)),
        ],
        out_shape=[
            jax.ShapeDtypeStruct((n, 1), jnp.float32),
            jax.ShapeDtypeStruct((n, d_hid), jnp.float32),
        ],
    )(degs, x, W1)

    p1 = edge_call(d_hid, g1)

    g2 = pl.pallas_call(
        _tc2_body,
        grid=(n // r,),
        in_specs=[
            pl.BlockSpec((NC, r, d_hid), lambda i: (0, i, 0)),
            pl.BlockSpec((r, d_hid), lambda i: (i, 0)),
            pl.BlockSpec((r, 1), lambda i: (i, 0)),
            pl.BlockSpec((1, d_hid), lambda i: (0, 0)),
            pl.BlockSpec((d_hid, d_out), lambda i: (0, 0)),
        ],
        out_specs=pl.BlockSpec((r, d_out), lambda i: (i, 0)),
        out_shape=jax.ShapeDtypeStruct((n, d_out), jnp.float32),
    )(p1, g1, dinv, b1.reshape(1, -1), W2)

    p2 = edge_call(d_out, g2)

    out = pl.pallas_call(
        _tc3_body,
        grid=(n // r,),
        in_specs=[
            pl.BlockSpec((NC, r, d_out), lambda i: (0, i, 0)),
            pl.BlockSpec((r, d_out), lambda i: (i, 0)),
            pl.BlockSpec((r, 1), lambda i: (i, 0)),
            pl.BlockSpec((1, d_out), lambda i: (0, 0)),
        ],
        out_specs=pl.BlockSpec((r, d_out), lambda i: (i, 0)),
        out_shape=jax.ShapeDtypeStruct((n, d_out), jnp.float32),
    )(p2, g2, dinv, b2.reshape(1, -1))

    return out


# speedup vs baseline: 18.5864x; 1.0256x over previous
import functools

import jax
import jax.numpy as jnp
from jax import lax
from jax.experimental import pallas as pl
from jax.experimental.pallas import tpu as pltpu
from jax.experimental.pallas import tpu_sc as plsc

NC = 2
NS = 16
NW = NC * NS
B = 128


def _largest_divisor_le(n, cap):
    for d in range(cap, 0, -1):
        if n % d == 0:
            return d
    return 1


def _deg_body(nrows, cpw, zr, dst_ref, degp_ref, acc, didx, ones_v, zbuf):
    rps = nrows // NS
    nz = rps // zr
    c = lax.axis_index("c")
    s = lax.axis_index("s")
    w = s * NC + c

    def oloop(i, carry):
        ones_v[i] = jnp.full((16,), 1.0, jnp.float32)
        return carry
    lax.fori_loop(0, B, oloop, 0)

    def zloop(i, carry):
        zbuf[i] = jnp.zeros((16,), jnp.float32)
        return carry
    lax.fori_loop(0, zr, zloop, 0)
    for t in range(nz):
        pltpu.sync_copy(zbuf, acc.at[pl.ds(s * rps + t * zr, zr)])
    plsc.subcore_barrier()

    def body(k, carry):
        pltpu.sync_copy(dst_ref.at[pl.ds((w * cpw + k) * B, B)], didx)
        pltpu.sync_copy(ones_v, acc.at[didx], add=True)
        return carry
    lax.fori_loop(0, cpw, body, 0)
    plsc.subcore_barrier()

    pltpu.sync_copy(acc.at[pl.ds(s * rps, rps)],
                    degp_ref.at[c, pl.ds(s * rps, rps)])


def _edge_body(nrows, d, cpw, zr, g_ref, src_ref, dst_ref, p_ref,
               acc, sidx0, didx0, sidx1, didx1, rows0, rows1, zbuf,
               sem0, sem1):
    rps = nrows // NS
    nz = rps // zr
    vpr = d // 16
    c = lax.axis_index("c")
    s = lax.axis_index("s")
    w = s * NC + c

    def zloop(t, carry):
        i = t // vpr
        j = t - i * vpr
        zbuf[i, pl.ds(j * 16, 16)] = jnp.zeros((16,), jnp.float32)
        return carry
    lax.fori_loop(0, zr * vpr, zloop, 0)
    for t in range(nz):
        pltpu.sync_copy(zbuf, acc.at[pl.ds(s * rps + t * zr, zr)])
    plsc.subcore_barrier()

    def body(j, carry):
        base0 = (w * cpw + 2 * j) * B
        ia = pltpu.async_copy(src_ref.at[pl.ds(base0, B)], sidx0, sem0)
        ib = pltpu.async_copy(dst_ref.at[pl.ds(base0, B)], didx0, sem0)
        ic = pltpu.async_copy(src_ref.at[pl.ds(base0 + B, B)], sidx1, sem1)
        id_ = pltpu.async_copy(dst_ref.at[pl.ds(base0 + B, B)], didx1, sem1)
        ia.wait()
        ib.wait()
        d0 = pltpu.async_copy(g_ref.at[sidx0], rows0, sem0)
        ic.wait()
        id_.wait()
        d1 = pltpu.async_copy(g_ref.at[sidx1], rows1, sem1)
        d0.wait()
        pltpu.sync_copy(rows0, acc.at[didx0], add=True)
        d1.wait()
        pltpu.sync_copy(rows1, acc.at[didx1], add=True)
        return carry
    lax.fori_loop(0, cpw // 2, body, 0)
    plsc.subcore_barrier()

    pltpu.sync_copy(acc.at[pl.ds(s * rps, rps)],
                    p_ref.at[c, pl.ds(s * rps, rps)])


def _tc1_body(degs_ref, x_ref, w_ref, dinv_ref, g_ref):
    deg = degs_ref[:, 0:1] + degs_ref[:, 1:2] + 1.0
    dinv = lax.rsqrt(deg)
    dinv_ref[...] = dinv
    g_ref[...] = jnp.dot(x_ref[...], w_ref[...],
                         preferred_element_type=jnp.float32,
                         precision=lax.Precision.HIGHEST) * dinv


def _tc2_body(p_ref, g1_ref, dinv_ref, b_ref, w_ref, g2_ref):
    ssum = p_ref[0] + p_ref[1] + g1_ref[...]
    h = jnp.maximum(dinv_ref[...] * ssum + b_ref[...], 0.0)
    g2_ref[...] = jnp.dot(h, w_ref[...],
                          preferred_element_type=jnp.float32,
                          precision=lax.Precision.HIGHEST) * dinv_ref[...]


def _tc3_body(p_ref, g2_ref, dinv_ref, b_ref, out_ref):
    out_ref[...] = dinv_ref[...] * (p_ref[0] + p_ref[1] + g2_ref[...]) \
        + b_ref[...]


def kernel(x, edge_index, W1, b1, W2, b2):
    n, d_in = x.shape
    d_hid = W1.shape[1]
    d_out = W2.shape[1]
    e = edge_index.shape[1]

    cpw = -(-e // (NW * B))
    cpw += cpw % 2
    pe = cpw * NW * B
    pad = pe - e
    nrows = ((n + 127) // 128 + 1) * 128
    ndump = nrows - n
    assert nrows % (NS * 8) == 0 and d_in % 16 == 0
    zr = _largest_divisor_le(nrows // NS, 128)
    zre = _largest_divisor_le(nrows // NS, 32)

    src = edge_index[0].astype(jnp.int32)
    dst = edge_index[1].astype(jnp.int32)
    ar = jnp.arange(pad, dtype=jnp.int32)
    src2 = jnp.concatenate([src, (ar * 997) % n])
    dst2 = jnp.concatenate([dst, n + (ar % ndump)])

    mesh = plsc.VectorSubcoreMesh(core_axis_name="c", subcore_axis_name="s")

    deg_call = pl.kernel(
        functools.partial(_deg_body, nrows, cpw, zr),
        out_type=jax.ShapeDtypeStruct((NC, nrows, 16), jnp.float32),
        mesh=mesh,
        scratch_types=[
            pltpu.VMEM_SHARED((nrows, 16), jnp.float32),
            pltpu.VMEM((B,), jnp.int32),
            pltpu.VMEM((B, 16), jnp.float32),
            pltpu.VMEM((zr, 16), jnp.float32),
        ],
    )
    degp = deg_call(dst2)
    degs = jnp.transpose(degp[:, :n, 0])

    def edge_call(d, g):
        return pl.kernel(
            functools.partial(_edge_body, nrows, d, cpw, zre),
            out_type=jax.ShapeDtypeStruct((NC, nrows, d), jnp.float32),
            mesh=mesh,
            scratch_types=[
                pltpu.VMEM_SHARED((nrows, d), jnp.float32),
                pltpu.VMEM((B,), jnp.int32),
                pltpu.VMEM((B,), jnp.int32),
                pltpu.VMEM((B,), jnp.int32),
                pltpu.VMEM((B,), jnp.int32),
                pltpu.VMEM((B, d), jnp.float32),
                pltpu.VMEM((B, d), jnp.float32),
                pltpu.VMEM((zre, d), jnp.float32),
                pltpu.SemaphoreType.DMA,
                pltpu.SemaphoreType.DMA,
            ],
        )(g, src2, dst2)

    r = n // 10
    dinv, g1 = pl.pallas_call(
        _tc1_body,
        grid=(n // r,),
        in_specs=[
            pl.BlockSpec((r, 2), lambda i: (i, 0)),
            pl.BlockSpec((r, d_in), lambda i: (i, 0)),
            pl.BlockSpec((d_in, d_hid), lambda i: (0, 0)),
        ],
        out_specs=[
            pl.BlockSpec((r, 1), lambda i: (i, 0)),
            pl.BlockSpec((r, d_hid), lambda i: (i, 0)),
        ],
        out_shape=[
            jax.ShapeDtypeStruct((n, 1), jnp.float32),
            jax.ShapeDtypeStruct((n, d_hid), jnp.float32),
        ],
    )(degs, x, W1)

    p1 = edge_call(d_hid, g1)

    g2 = pl.pallas_call(
        _tc2_body,
        grid=(n // r,),
        in_specs=[
            pl.BlockSpec((NC, r, d_hid), lambda i: (0, i, 0)),
            pl.BlockSpec((r, d_hid), lambda i: (i, 0)),
            pl.BlockSpec((r, 1), lambda i: (i, 0)),
            pl.BlockSpec((1, d_hid), lambda i: (0, 0)),
            pl.BlockSpec((d_hid, d_out), lambda i: (0, 0)),
        ],
        out_specs=pl.BlockSpec((r, d_out), lambda i: (i, 0)),
        out_shape=jax.ShapeDtypeStruct((n, d_out), jnp.float32),
    )(p1, g1, dinv, b1.reshape(1, -1), W2)

    p2 = edge_call(d_out, g2)

    out = pl.pallas_call(
        _tc3_body,
        grid=(n // r,),
        in_specs=[
            pl.BlockSpec((NC, r, d_out), lambda i: (0, i, 0)),
            pl.BlockSpec((r, d_out), lambda i: (i, 0)),
            pl.BlockSpec((r, 1), lambda i: (i, 0)),
            pl.BlockSpec((1, d_out), lambda i: (0, 0)),
        ],
        out_specs=pl.BlockSpec((r, d_out), lambda i: (i, 0)),
        out_shape=jax.ShapeDtypeStruct((n, d_out), jnp.float32),
    )(p2, g2, dinv, b2.reshape(1, -1))

    return out
